# baseline (speedup 1.0000x reference)
import functools

import jax
import jax.numpy as jnp
import numpy as np
from jax import lax
from jax.experimental import pallas as pl
from jax.experimental.pallas import tpu as pltpu
from jax.experimental.pallas import tpu_sc as plsc

_USE_TEXT_QUERY = True
_USE_TEXT_CANDS = True
_N_CANDS = 5
_TEXT_QUESTION = 0
_TEXT_EMBEDDING = 1
_VISUAL_EMBEDDING = 3


def _static_class_ids(L: int) -> np.ndarray:
    ids = []
    if _USE_TEXT_QUERY:
        ids.append(_TEXT_QUESTION)
    if _USE_TEXT_CANDS:
        ids.extend([_TEXT_EMBEDDING] * _N_CANDS)
    n_frames = L - len(ids)
    ids.extend([_VISUAL_EMBEDDING] * n_frames)
    return np.asarray(ids, dtype=np.int32)


@functools.lru_cache(maxsize=None)
def _make_sc_gather(L: int, D: int):
    info = plsc.get_sparse_core_info()
    NC, NS = info.num_cores, info.num_subcores
    NW = NC * NS
    assert L % NW == 0 and (L // NW) % 8 == 0
    b_per_w = L // NW
    mesh = plsc.VectorSubcoreMesh(core_axis_name="c", subcore_axis_name="s")

    @functools.partial(
        pl.kernel,
        mesh=mesh,
        out_type=jax.ShapeDtypeStruct((L, D), jnp.float32),
        scratch_types=[
            pltpu.VMEM((b_per_w,), jnp.int32),
            pltpu.VMEM((b_per_w, D), jnp.float32),
            pltpu.SemaphoreType.DMA,
        ],
    )
    def k(table_hbm, idx_hbm, out_hbm, idx_v, rows_v, sem):
        wid = lax.axis_index("s") * NC + lax.axis_index("c")
        base = wid * b_per_w
        pltpu.sync_copy(idx_hbm.at[pl.ds(base, b_per_w)], idx_v)
        pltpu.async_copy(table_hbm.at[idx_v], rows_v, sem).wait()
        pltpu.sync_copy(rows_v, out_hbm.at[pl.ds(base, b_per_w)])

    return k


def kernel(x, table):
    L, N, D = x.shape
    idx = jnp.asarray(_static_class_ids(L))
    out = _make_sc_gather(L, D)(table, idx)
    return out[:, None, :]


# speedup vs baseline: 2.3963x; 2.3963x over previous
import functools

import jax
import jax.numpy as jnp
import numpy as np
from jax import lax
from jax.experimental import pallas as pl
from jax.experimental.pallas import tpu as pltpu
from jax.experimental.pallas import tpu_sc as plsc

_USE_TEXT_QUERY = True
_USE_TEXT_CANDS = True
_N_CANDS = 5
_TEXT_QUESTION = 0
_TEXT_EMBEDDING = 1
_VISUAL_EMBEDDING = 3


def _static_class_ids(L: int) -> np.ndarray:
    ids = []
    if _USE_TEXT_QUERY:
        ids.append(_TEXT_QUESTION)
    if _USE_TEXT_CANDS:
        ids.extend([_TEXT_EMBEDDING] * _N_CANDS)
    n_frames = L - len(ids)
    ids.extend([_VISUAL_EMBEDDING] * n_frames)
    return np.asarray(ids, dtype=np.int32)


@functools.lru_cache(maxsize=None)
def _make_sc_gather(L: int, D: int):
    info = plsc.get_sparse_core_info()
    NC, NS = info.num_cores, info.num_subcores
    NW = NC * NS
    assert L % NW == 0 and (L // NW) % 8 == 0
    b_per_w = L // NW
    mesh = plsc.VectorSubcoreMesh(core_axis_name="c", subcore_axis_name="s")

    TILE = 8
    n_blocks = b_per_w // TILE

    @functools.partial(
        pl.kernel,
        mesh=mesh,
        out_type=jax.ShapeDtypeStruct((L, D), jnp.float32),
        scratch_types=[
            pltpu.VMEM((TILE, D), jnp.float32),
            pltpu.SemaphoreType.DMA,
        ],
    )
    def k(table_hbm, out_hbm, buf_v, sem):
        wid = lax.axis_index("s") * NC + lax.axis_index("c")
        base = wid * b_per_w
        vis = table_hbm.at[pl.ds(_VISUAL_EMBEDDING, 1)]

        fills = [
            pltpu.async_copy(vis, buf_v.at[pl.ds(r, 1)], sem)
            for r in range(TILE)
        ]
        for f in fills:
            f.wait()

        @pl.when(wid == 0)
        def _prefix_block():
            fixes = [
                pltpu.async_copy(
                    table_hbm.at[pl.ds(_TEXT_QUESTION, 1)],
                    buf_v.at[pl.ds(0, 1)],
                    sem,
                )
            ] + [
                pltpu.async_copy(
                    table_hbm.at[pl.ds(_TEXT_EMBEDDING, 1)],
                    buf_v.at[pl.ds(r, 1)],
                    sem,
                )
                for r in range(1, 1 + _N_CANDS)
            ]
            for f in fixes:
                f.wait()
            pltpu.sync_copy(buf_v, out_hbm.at[pl.ds(base, TILE)])
            restores = [
                pltpu.async_copy(vis, buf_v.at[pl.ds(r, 1)], sem)
                for r in range(1 + _N_CANDS)
            ]
            for f in restores:
                f.wait()

        @pl.when(wid != 0)
        def _block0():
            pltpu.sync_copy(buf_v, out_hbm.at[pl.ds(base, TILE)])

        writes = [
            pltpu.async_copy(
                buf_v,
                out_hbm.at[pl.ds(base + b * TILE, TILE)],
                sem,
            )
            for b in range(1, n_blocks)
        ]
        for w in writes:
            w.wait()

    return k


def kernel(x, table):
    L, N, D = x.shape
    out = _make_sc_gather(L, D)(table)
    return out[:, None, :]


# speedup vs baseline: 3.3696x; 1.4062x over previous
import functools

import jax
import jax.numpy as jnp
import numpy as np
from jax import lax
from jax.experimental import pallas as pl
from jax.experimental.pallas import tpu as pltpu
from jax.experimental.pallas import tpu_sc as plsc

_USE_TEXT_QUERY = True
_USE_TEXT_CANDS = True
_N_CANDS = 5
_TEXT_QUESTION = 0
_TEXT_EMBEDDING = 1
_VISUAL_EMBEDDING = 3


def _prefix_ids() -> list:
    ids = []
    if _USE_TEXT_QUERY:
        ids.append(_TEXT_QUESTION)
    if _USE_TEXT_CANDS:
        ids.extend([_TEXT_EMBEDDING] * _N_CANDS)
    return ids


@functools.lru_cache(maxsize=None)
def _make_sc_fill(L: int, D: int):
    info = plsc.get_sparse_core_info()
    NC, NS = info.num_cores, info.num_subcores
    NW = NC * NS
    assert L % NW == 0 and (L // NW) % 8 == 0
    b_per_w = L // NW
    TILE = 8
    n_blocks = b_per_w // TILE
    prefix = _prefix_ids()
    assert len(prefix) < TILE
    block0_ids = prefix + [_VISUAL_EMBEDDING] * (TILE - len(prefix))
    mesh = plsc.VectorSubcoreMesh(core_axis_name="c", subcore_axis_name="s")

    @functools.partial(
        pl.kernel,
        mesh=mesh,
        out_type=jax.ShapeDtypeStruct((L, 1, D), jnp.float32),
        scratch_types=[
            pltpu.VMEM((TILE, 1, D), jnp.float32),
            pltpu.VMEM((TILE, 1, D), jnp.float32),
            pltpu.SemaphoreType.DMA,
        ],
    )
    def k(table_hbm, out_hbm, buf_v, buf0_v, sem):
        wid = lax.axis_index("s") * NC + lax.axis_index("c")
        base = wid * b_per_w
        vis = table_hbm.at[pl.ds(_VISUAL_EMBEDDING, 1)]

        fills = [
            pltpu.async_copy(vis, buf_v.at[pl.ds(r, 1), 0], sem)
            for r in range(TILE)
        ]

        @pl.when(wid == 0)
        def _fill_prefix():
            f0 = [
                pltpu.async_copy(
                    table_hbm.at[pl.ds(i, 1)], buf0_v.at[pl.ds(r, 1), 0], sem
                )
                for r, i in enumerate(block0_ids)
            ]
            for f in f0:
                f.wait()

        for f in fills:
            f.wait()

        @pl.when(wid == 0)
        def _write_block0():
            pltpu.sync_copy(buf0_v, out_hbm.at[pl.ds(base, TILE)])

        @pl.when(wid != 0)
        def _write_block0_common():
            pltpu.sync_copy(buf_v, out_hbm.at[pl.ds(base, TILE)])

        writes = [
            pltpu.async_copy(
                buf_v, out_hbm.at[pl.ds(base + b * TILE, TILE)], sem
            )
            for b in range(1, n_blocks)
        ]
        for w in writes:
            w.wait()

    return k


def kernel(x, table):
    L, N, D = x.shape
    return _make_sc_fill(L, D)(table)
